# baseline (speedup 1.0000x reference)
import functools

import jax
import jax.numpy as jnp
from jax import lax
from jax.experimental import pallas as pl
from jax.experimental.pallas import tpu as pltpu
from jax.experimental.pallas import tpu_sc as plsc

N = 10000
E = 320000
K = 10

NCORE = 2
NTEC = 16
DH = 64
C = 128
KB = 2
NBUF = 2
NGRPS = 80
EPT = NGRPS * KB * C
NIB = 4
HGRPS = 20
NPT = 640
NROWCH = 128
NNCH = NPT // NROWCH
NPAD = 10240
QN = DH // 16


def _edge_pipeline(eidx_s, ibuf, rows_v, t_c, t_h, agg_sh, isem, gsem, ssem):
    for j in range(3):
        pltpu.async_copy(eidx_s.at[j], ibuf.at[j], isem.at[j])
    pltpu.make_async_copy(eidx_s.at[0], ibuf.at[0], isem.at[0]).wait()
    pltpu.async_copy(t_h.at[ibuf.at[0, 0]], rows_v.at[0], gsem.at[0])

    @pl.loop(0, NGRPS - 1)
    def _(j):
        b = j % NBUF
        nb = (j + 1) % NBUF
        pltpu.make_async_copy(
            t_c.at[ibuf.at[b, 0]], rows_v.at[b], gsem.at[b]).wait()
        pltpu.async_copy(
            rows_v.at[b], agg_sh.at[ibuf.at[j % NIB, 1]], ssem.at[b],
            add=True)

        @pl.when(j > 0)
        def _():
            pltpu.make_async_copy(
                rows_v.at[nb], agg_sh.at[ibuf.at[0, 1]], ssem.at[nb]).wait()

        @pl.when(j + 3 < NGRPS)
        def _():
            pltpu.async_copy(eidx_s.at[j + 3], ibuf.at[(j + 3) % NIB],
                             isem.at[(j + 3) % NIB])

        pltpu.make_async_copy(
            eidx_s.at[0], ibuf.at[0], isem.at[(j + 1) % NIB]).wait()

        @pl.when(j + 1 < HGRPS)
        def _():
            pltpu.async_copy(
                t_h.at[ibuf.at[(j + 1) % NIB, 0]], rows_v.at[nb],
                gsem.at[nb])

        @pl.when(j + 1 >= HGRPS)
        def _():
            pltpu.async_copy(
                t_c.at[ibuf.at[(j + 1) % NIB, 0]], rows_v.at[nb],
                gsem.at[nb])

    jf = NGRPS - 1
    bf = jf % NBUF
    pltpu.make_async_copy(
        t_c.at[ibuf.at[bf, 0]], rows_v.at[bf], gsem.at[bf]).wait()
    pltpu.async_copy(
        rows_v.at[bf], agg_sh.at[ibuf.at[jf % NIB, 1]], ssem.at[bf],
        add=True)
    pltpu.make_async_copy(
        rows_v.at[1 - bf], agg_sh.at[ibuf.at[0, 1]], ssem.at[1 - bf]).wait()
    pltpu.make_async_copy(
        rows_v.at[bf], agg_sh.at[ibuf.at[0, 1]], ssem.at[bf]).wait()


def _make_body(with_deg):
    def body(*refs):
        if with_deg:
            (z_h, eidx, zeros64,
             out_h, t_hbm, bt_hbm,
             agg_sh, t_c, ibuf, rows_v, b_t, isem, gsem, ssem, dsem) = refs
        else:
            (z_h, eidx, zeros64, bt_hbm,
             out_h, t_hbm,
             agg_sh, t_c, ibuf, rows_v, b_t, isem, gsem, ssem, dsem) = refs
        c = lax.axis_index("c")
        s = lax.axis_index("s")
        t_h = t_hbm.at[c]
        eidx_s = eidx.at[s]
        aggb = rows_v.at[0, pl.ds(0, NROWCH), :]
        tb = rows_v.at[0, pl.ds(NROWCH, NROWCH), :]
        zb = rows_v.at[1, pl.ds(0, NROWCH), :]

        @pl.loop(0, NNCH)
        def _(ch):
            row0 = pl.multiple_of(s * NPT + ch * NROWCH, 8)
            pltpu.sync_copy(zeros64.at[pl.ds(row0, NROWCH), :],
                            agg_sh.at[pl.ds(row0, NROWCH), :])

        if with_deg:
            ones16v = jnp.ones((16,), jnp.float32)

            @pl.loop(0, KB * C)
            def _(r):
                for q in range(QN):
                    rows_v[0, r, pl.ds(q * 16, 16)] = ones16v

            plsc.subcore_barrier()

            @pl.loop(0, NGRPS // NIB)
            def _(u):
                pd = []
                for q in range(NIB):
                    pd.append(pltpu.async_copy(
                        eidx_s.at[u * NIB + q], ibuf.at[q], isem.at[q]))
                sd = []
                for q in range(NIB):
                    pd[q].wait()
                    sd.append(pltpu.async_copy(
                        rows_v.at[0], agg_sh.at[ibuf.at[q, 1]], dsem,
                        add=True))
                for d in sd:
                    d.wait()

            plsc.subcore_barrier()

            @pl.loop(0, NNCH)
            def _(ch):
                row0 = pl.multiple_of(s * NPT + ch * NROWCH, 8)
                pltpu.sync_copy(agg_sh.at[pl.ds(row0, NROWCH), :], aggb)

                @pl.loop(0, NROWCH)
                def _(r):
                    d = aggb[r, pl.ds(0, 16)] + 1.0
                    sq = jnp.where(d < 16.0, 1.0,
                                   jnp.where(d < 256.0, 4.0,
                                             jnp.where(d < 4096.0, 16.0,
                                                       64.0)))
                    sq = jnp.where(d < 65536.0, sq, 256.0)
                    for _ in range(8):
                        sq = 0.5 * (sq + d / sq)
                    b_t[ch * NROWCH + r, :] = 0.1 / sq

            pltpu.sync_copy(b_t, bt_hbm.at[c, s])
        else:
            pltpu.sync_copy(bt_hbm.at[c, s], b_t)

        @pl.loop(0, NNCH)
        def _(ch):
            row0 = pl.multiple_of(s * NPT + ch * NROWCH, 8)
            pltpu.sync_copy(z_h.at[c, s, pl.ds(ch * NROWCH, NROWCH), :], zb)

            @pl.loop(0, NROWCH)
            def _(r):
                cv = 10.0 * b_t[ch * NROWCH + r, :]
                for q in range(QN):
                    sl = pl.ds(q * 16, 16)
                    tb[r, sl] = cv * zb[r, sl]

            pltpu.sync_copy(tb, t_c.at[pl.ds(row0, NROWCH), :])
            pltpu.sync_copy(tb, t_h.at[pl.ds(row0, NROWCH), :])
            pltpu.sync_copy(zeros64.at[pl.ds(row0, NROWCH), :],
                            agg_sh.at[pl.ds(row0, NROWCH), :])

        plsc.subcore_barrier()

        @pl.loop(0, K)
        def _(k):
            _edge_pipeline(eidx_s, ibuf, rows_v, t_c, t_h, agg_sh,
                           isem, gsem, ssem)
            plsc.subcore_barrier()

            @pl.loop(0, NNCH)
            def _(ch):
                row0 = pl.multiple_of(s * NPT + ch * NROWCH, 8)
                d1 = pltpu.async_copy(
                    agg_sh.at[pl.ds(row0, NROWCH), :], aggb, gsem.at[0])
                d2 = pltpu.async_copy(
                    t_c.at[pl.ds(row0, NROWCH), :], tb, gsem.at[1])
                d3 = pltpu.async_copy(
                    z_h.at[c, s, pl.ds(ch * NROWCH, NROWCH), :], zb,
                    ssem.at[0])
                d1.wait()
                d2.wait()
                d3.wait()

                @pl.loop(0, NROWCH)
                def _(r):
                    bv = b_t[ch * NROWCH + r, :]
                    av = 90.0 * bv * bv
                    for q in range(QN):
                        sl = pl.ds(q * 16, 16)
                        aggb[r, sl] = (av * (aggb[r, sl] + tb[r, sl])
                                       + bv * zb[r, sl])

                d4 = pltpu.async_copy(
                    aggb, t_c.at[pl.ds(row0, NROWCH), :], gsem.at[0])
                d5 = pltpu.async_copy(
                    zeros64.at[pl.ds(row0, NROWCH), :],
                    agg_sh.at[pl.ds(row0, NROWCH), :], gsem.at[1])
                d6 = pltpu.async_copy(
                    aggb, t_h.at[pl.ds(row0, NROWCH), :], ssem.at[0])
                d4.wait()
                d5.wait()
                d6.wait()

            plsc.subcore_barrier()

        @pl.loop(0, NNCH)
        def _(ch):
            row0 = pl.multiple_of(s * NPT + ch * NROWCH, 8)
            pltpu.sync_copy(t_c.at[pl.ds(row0, NROWCH), :], tb)

            @pl.loop(0, NROWCH)
            def _(r):
                dv = 1.0 / (10.0 * b_t[ch * NROWCH + r, :])
                for q in range(QN):
                    sl = pl.ds(q * 16, 16)
                    tb[r, sl] = dv * tb[r, sl]

            pltpu.sync_copy(tb, out_h.at[c, s, pl.ds(ch * NROWCH, NROWCH), :])

    return body


_SCRATCH = [
    pltpu.VMEM_SHARED((NPAD, DH), jnp.float32),
    pltpu.VMEM_SHARED((NPAD, DH), jnp.float32),
    pltpu.VMEM((NIB, 2, KB * C), jnp.int32),
    pltpu.VMEM((NBUF, KB * C, DH), jnp.float32),
    pltpu.VMEM((NPT, 16), jnp.float32),
    pltpu.SemaphoreType.DMA((NIB,)),
    pltpu.SemaphoreType.DMA((NBUF,)),
    pltpu.SemaphoreType.DMA((NBUF,)),
    pltpu.SemaphoreType.DMA,
]

_sc_prop1 = functools.partial(
    pl.kernel,
    out_type=(
        jax.ShapeDtypeStruct((NCORE, NTEC, NPT, DH), jnp.float32),
        jax.ShapeDtypeStruct((NCORE, NPAD, DH), jnp.float32),
        jax.ShapeDtypeStruct((NCORE, NTEC, NPT, 16), jnp.float32),
    ),
    mesh=plsc.VectorSubcoreMesh(core_axis_name="c", subcore_axis_name="s"),
    compiler_params=pltpu.CompilerParams(use_tc_tiling_on_sc=False),
    scratch_types=_SCRATCH,
)(_make_body(True))

_sc_prop2 = functools.partial(
    pl.kernel,
    out_type=(
        jax.ShapeDtypeStruct((NCORE, NTEC, NPT, DH), jnp.float32),
        jax.ShapeDtypeStruct((NCORE, NPAD, DH), jnp.float32),
    ),
    mesh=plsc.VectorSubcoreMesh(core_axis_name="c", subcore_axis_name="s"),
    compiler_params=pltpu.CompilerParams(use_tc_tiling_on_sc=False),
    scratch_types=_SCRATCH,
)(_make_body(False))


def _mlp_kernel(u_ref, w1_ref, b1_ref, w2_ref, b2_ref, o_ref):
    h = jnp.maximum(u_ref[...] @ w1_ref[...] + b1_ref[...], 0.0)
    o_ref[...] = h @ w2_ref[...] + b2_ref[...]


def _mlp(u, W1, b1, W2, b2, block_rows=1000):
    n, d_in = u.shape
    d_mid = W1.shape[1]
    d_out = W2.shape[1]
    return pl.pallas_call(
        _mlp_kernel,
        grid=(n // block_rows,),
        in_specs=[
            pl.BlockSpec((block_rows, d_in), lambda i: (i, 0)),
            pl.BlockSpec((d_in, d_mid), lambda i: (0, 0)),
            pl.BlockSpec((d_mid,), lambda i: (0,)),
            pl.BlockSpec((d_mid, d_out), lambda i: (0, 0)),
            pl.BlockSpec((d_out,), lambda i: (0,)),
        ],
        out_specs=pl.BlockSpec((block_rows, d_out), lambda i: (i, 0)),
        out_shape=jax.ShapeDtypeStruct((n, d_out), jnp.float32),
    )(u, W1, b1, W2, b2)


def _to_halves(z):
    zp = jnp.concatenate(
        [z, jnp.zeros((NPAD - N, z.shape[1]), jnp.float32)])
    return zp.reshape(NTEC, NPT, NCORE, DH).transpose(2, 0, 1, 3)


def _from_halves(z_h):
    return z_h.transpose(1, 2, 0, 3).reshape(NPAD, NCORE * DH)[:N]


def kernel(x, edge_index, W1, b1, W2, b2):
    src = edge_index[0].astype(jnp.int32)
    dst = edge_index[1].astype(jnp.int32)
    pad = jnp.full((NTEC * EPT - E,), N, jnp.int32)
    src_p = jnp.concatenate([src, pad]).reshape(NTEC, NGRPS, KB * C)
    dst_p = jnp.concatenate([dst, pad]).reshape(NTEC, NGRPS, KB * C)
    eidx = jnp.stack([src_p, dst_p], axis=2)

    zeros64 = jnp.zeros((NPAD, DH), jnp.float32)

    u_h, _, bt = _sc_prop1(_to_halves(x), eidx, zeros64)
    z2 = _mlp(_from_halves(u_h), W1, b1, W2, b2)
    out_h, _ = _sc_prop2(_to_halves(z2), eidx, zeros64, bt)
    return _from_halves(out_h)


# speedup vs baseline: 1.0088x; 1.0088x over previous
import functools

import jax
import jax.numpy as jnp
from jax import lax
from jax.experimental import pallas as pl
from jax.experimental.pallas import tpu as pltpu
from jax.experimental.pallas import tpu_sc as plsc

N = 10000
E = 320000
K = 10

NCORE = 2
NTEC = 16
DH = 64
C = 128
KB = 2
NBUF = 2
NGRPS = 80
EPT = NGRPS * KB * C
NIB = 4
NPT = 640
NROWCH = 128
NNCH = NPT // NROWCH
NPAD = 10240
QN = DH // 16


def _edge_pipeline(eidx_s, ibuf, rows_v, t_c, agg_sh, isem, gsem, ssem):
    for j in range(3):
        pltpu.async_copy(eidx_s.at[j], ibuf.at[j], isem.at[j])
    pltpu.make_async_copy(eidx_s.at[0], ibuf.at[0], isem.at[0]).wait()
    pltpu.async_copy(t_c.at[ibuf.at[0, 0]], rows_v.at[0], gsem.at[0])

    @pl.loop(0, NGRPS - 1)
    def _(j):
        b = j % NBUF
        nb = (j + 1) % NBUF
        pltpu.make_async_copy(
            t_c.at[ibuf.at[b, 0]], rows_v.at[b], gsem.at[b]).wait()
        pltpu.async_copy(
            rows_v.at[b], agg_sh.at[ibuf.at[j % NIB, 1]], ssem.at[b],
            add=True)

        @pl.when(j > 0)
        def _():
            pltpu.make_async_copy(
                rows_v.at[nb], agg_sh.at[ibuf.at[0, 1]], ssem.at[nb]).wait()

        @pl.when(j + 3 < NGRPS)
        def _():
            pltpu.async_copy(eidx_s.at[j + 3], ibuf.at[(j + 3) % NIB],
                             isem.at[(j + 3) % NIB])

        pltpu.make_async_copy(
            eidx_s.at[0], ibuf.at[0], isem.at[(j + 1) % NIB]).wait()
        pltpu.async_copy(
            t_c.at[ibuf.at[(j + 1) % NIB, 0]], rows_v.at[nb], gsem.at[nb])

    jf = NGRPS - 1
    bf = jf % NBUF
    pltpu.make_async_copy(
        t_c.at[ibuf.at[bf, 0]], rows_v.at[bf], gsem.at[bf]).wait()
    pltpu.async_copy(
        rows_v.at[bf], agg_sh.at[ibuf.at[jf % NIB, 1]], ssem.at[bf],
        add=True)
    pltpu.make_async_copy(
        rows_v.at[1 - bf], agg_sh.at[ibuf.at[0, 1]], ssem.at[1 - bf]).wait()
    pltpu.make_async_copy(
        rows_v.at[bf], agg_sh.at[ibuf.at[0, 1]], ssem.at[bf]).wait()


def _make_body(with_deg):
    def body(*refs):
        if with_deg:
            (z_h, eidx, zeros64,
             out_h, bt_hbm,
             agg_sh, t_c, ibuf, rows_v, b_t, isem, gsem, ssem, dsem) = refs
        else:
            (z_h, eidx, zeros64, bt_hbm,
             out_h,
             agg_sh, t_c, ibuf, rows_v, b_t, isem, gsem, ssem, dsem) = refs
        c = lax.axis_index("c")
        s = lax.axis_index("s")
        eidx_s = eidx.at[s]
        aggb = rows_v.at[0, pl.ds(0, NROWCH), :]
        tb = rows_v.at[0, pl.ds(NROWCH, NROWCH), :]
        zb = rows_v.at[1, pl.ds(0, NROWCH), :]

        @pl.loop(0, NNCH)
        def _(ch):
            row0 = pl.multiple_of(s * NPT + ch * NROWCH, 8)
            pltpu.sync_copy(zeros64.at[pl.ds(row0, NROWCH), :],
                            agg_sh.at[pl.ds(row0, NROWCH), :])

        if with_deg:
            ones16v = jnp.ones((16,), jnp.float32)

            @pl.loop(0, KB * C)
            def _(r):
                for q in range(QN):
                    rows_v[0, r, pl.ds(q * 16, 16)] = ones16v

            plsc.subcore_barrier()

            @pl.loop(0, NGRPS // NIB)
            def _(u):
                pd = []
                for q in range(NIB):
                    pd.append(pltpu.async_copy(
                        eidx_s.at[u * NIB + q], ibuf.at[q], isem.at[q]))
                sd = []
                for q in range(NIB):
                    pd[q].wait()
                    sd.append(pltpu.async_copy(
                        rows_v.at[0], agg_sh.at[ibuf.at[q, 1]], dsem,
                        add=True))
                for d in sd:
                    d.wait()

            plsc.subcore_barrier()

            @pl.loop(0, NNCH)
            def _(ch):
                row0 = pl.multiple_of(s * NPT + ch * NROWCH, 8)
                pltpu.sync_copy(agg_sh.at[pl.ds(row0, NROWCH), :], aggb)

                @pl.loop(0, NROWCH)
                def _(r):
                    d = aggb[r, pl.ds(0, 16)] + 1.0
                    sq = jnp.where(d < 16.0, 1.0,
                                   jnp.where(d < 256.0, 4.0,
                                             jnp.where(d < 4096.0, 16.0,
                                                       64.0)))
                    sq = jnp.where(d < 65536.0, sq, 256.0)
                    for _ in range(8):
                        sq = 0.5 * (sq + d / sq)
                    b_t[ch * NROWCH + r, :] = 0.1 / sq

            pltpu.sync_copy(b_t, bt_hbm.at[c, s])
        else:
            pltpu.sync_copy(bt_hbm.at[c, s], b_t)

        @pl.loop(0, NNCH)
        def _(ch):
            row0 = pl.multiple_of(s * NPT + ch * NROWCH, 8)
            pltpu.sync_copy(z_h.at[c, s, pl.ds(ch * NROWCH, NROWCH), :], zb)

            @pl.loop(0, NROWCH)
            def _(r):
                cv = 10.0 * b_t[ch * NROWCH + r, :]
                for q in range(QN):
                    sl = pl.ds(q * 16, 16)
                    tb[r, sl] = cv * zb[r, sl]

            pltpu.sync_copy(tb, t_c.at[pl.ds(row0, NROWCH), :])
            pltpu.sync_copy(zeros64.at[pl.ds(row0, NROWCH), :],
                            agg_sh.at[pl.ds(row0, NROWCH), :])

        plsc.subcore_barrier()

        @pl.loop(0, K)
        def _(k):
            _edge_pipeline(eidx_s, ibuf, rows_v, t_c, agg_sh,
                           isem, gsem, ssem)
            plsc.subcore_barrier()

            zero16v = jnp.zeros((16,), jnp.float32)

            @pl.loop(0, NNCH)
            def _(ch):
                row0 = pl.multiple_of(s * NPT + ch * NROWCH, 8)
                d1 = pltpu.async_copy(
                    agg_sh.at[pl.ds(row0, NROWCH), :], aggb, gsem.at[0])
                d2 = pltpu.async_copy(
                    t_c.at[pl.ds(row0, NROWCH), :], tb, gsem.at[1])
                d3 = pltpu.async_copy(
                    z_h.at[c, s, pl.ds(ch * NROWCH, NROWCH), :], zb,
                    ssem.at[0])
                d1.wait()
                d2.wait()
                d3.wait()

                @pl.loop(0, NROWCH, unroll=2)
                def _(r):
                    bv = b_t[ch * NROWCH + r, :]
                    av = 90.0 * bv * bv
                    for q in range(QN):
                        sl = pl.ds(q * 16, 16)
                        tb[r, sl] = (av * (aggb[r, sl] + tb[r, sl])
                                     + bv * zb[r, sl])
                        aggb[r, sl] = zero16v

                d4 = pltpu.async_copy(
                    tb, t_c.at[pl.ds(row0, NROWCH), :], gsem.at[0])
                d5 = pltpu.async_copy(
                    aggb, agg_sh.at[pl.ds(row0, NROWCH), :], gsem.at[1])
                d4.wait()
                d5.wait()

            plsc.subcore_barrier()

        @pl.loop(0, NNCH)
        def _(ch):
            row0 = pl.multiple_of(s * NPT + ch * NROWCH, 8)
            pltpu.sync_copy(t_c.at[pl.ds(row0, NROWCH), :], tb)

            @pl.loop(0, NROWCH)
            def _(r):
                dv = 1.0 / (10.0 * b_t[ch * NROWCH + r, :])
                for q in range(QN):
                    sl = pl.ds(q * 16, 16)
                    tb[r, sl] = dv * tb[r, sl]

            pltpu.sync_copy(tb, out_h.at[c, s, pl.ds(ch * NROWCH, NROWCH), :])

    return body


_SCRATCH = [
    pltpu.VMEM_SHARED((NPAD, DH), jnp.float32),
    pltpu.VMEM_SHARED((NPAD, DH), jnp.float32),
    pltpu.VMEM((NIB, 2, KB * C), jnp.int32),
    pltpu.VMEM((NBUF, KB * C, DH), jnp.float32),
    pltpu.VMEM((NPT, 16), jnp.float32),
    pltpu.SemaphoreType.DMA((NIB,)),
    pltpu.SemaphoreType.DMA((NBUF,)),
    pltpu.SemaphoreType.DMA((NBUF,)),
    pltpu.SemaphoreType.DMA,
]

_sc_prop1 = functools.partial(
    pl.kernel,
    out_type=(
        jax.ShapeDtypeStruct((NCORE, NTEC, NPT, DH), jnp.float32),
        jax.ShapeDtypeStruct((NCORE, NTEC, NPT, 16), jnp.float32),
    ),
    mesh=plsc.VectorSubcoreMesh(core_axis_name="c", subcore_axis_name="s"),
    compiler_params=pltpu.CompilerParams(use_tc_tiling_on_sc=False),
    scratch_types=_SCRATCH,
)(_make_body(True))

_sc_prop2 = functools.partial(
    pl.kernel,
    out_type=jax.ShapeDtypeStruct((NCORE, NTEC, NPT, DH), jnp.float32),
    mesh=plsc.VectorSubcoreMesh(core_axis_name="c", subcore_axis_name="s"),
    compiler_params=pltpu.CompilerParams(use_tc_tiling_on_sc=False),
    scratch_types=_SCRATCH,
)(_make_body(False))


def _mlp_kernel(u_ref, w1_ref, b1_ref, w2_ref, b2_ref, o_ref):
    h = jnp.maximum(u_ref[...] @ w1_ref[...] + b1_ref[...], 0.0)
    o_ref[...] = h @ w2_ref[...] + b2_ref[...]


def _mlp(u, W1, b1, W2, b2, block_rows=1000):
    n, d_in = u.shape
    d_mid = W1.shape[1]
    d_out = W2.shape[1]
    return pl.pallas_call(
        _mlp_kernel,
        grid=(n // block_rows,),
        in_specs=[
            pl.BlockSpec((block_rows, d_in), lambda i: (i, 0)),
            pl.BlockSpec((d_in, d_mid), lambda i: (0, 0)),
            pl.BlockSpec((d_mid,), lambda i: (0,)),
            pl.BlockSpec((d_mid, d_out), lambda i: (0, 0)),
            pl.BlockSpec((d_out,), lambda i: (0,)),
        ],
        out_specs=pl.BlockSpec((block_rows, d_out), lambda i: (i, 0)),
        out_shape=jax.ShapeDtypeStruct((n, d_out), jnp.float32),
    )(u, W1, b1, W2, b2)


def _to_halves(z):
    zp = jnp.concatenate(
        [z, jnp.zeros((NPAD - N, z.shape[1]), jnp.float32)])
    return zp.reshape(NTEC, NPT, NCORE, DH).transpose(2, 0, 1, 3)


def _from_halves(z_h):
    return z_h.transpose(1, 2, 0, 3).reshape(NPAD, NCORE * DH)[:N]


def kernel(x, edge_index, W1, b1, W2, b2):
    src = edge_index[0].astype(jnp.int32)
    dst = edge_index[1].astype(jnp.int32)
    pad = jnp.full((NTEC * EPT - E,), N, jnp.int32)
    src_p = jnp.concatenate([src, pad]).reshape(NTEC, NGRPS, KB * C)
    dst_p = jnp.concatenate([dst, pad]).reshape(NTEC, NGRPS, KB * C)
    eidx = jnp.stack([src_p, dst_p], axis=2)

    zeros64 = jnp.zeros((NPAD, DH), jnp.float32)

    u_h, bt = _sc_prop1(_to_halves(x), eidx, zeros64)
    z2 = _mlp(_from_halves(u_h), W1, b1, W2, b2)
    out_h = _sc_prop2(_to_halves(z2), eidx, zeros64, bt)
    return _from_halves(out_h)


# speedup vs baseline: 1.0188x; 1.0099x over previous
import functools

import jax
import jax.numpy as jnp
from jax import lax
from jax.experimental import pallas as pl
from jax.experimental.pallas import tpu as pltpu
from jax.experimental.pallas import tpu_sc as plsc

N = 10000
E = 320000
K = 10

NCORE = 2
NTEC = 16
DH = 64
C = 128
KB = 2
NBUF = 2
NGRPS = 80
EPT = NGRPS * KB * C
NIB = 4
NPT = 640
NROWCH = 128
NNCH = NPT // NROWCH
NPAD = 10240
QN = DH // 16


def _edge_pipeline(eidx_s, ibuf, rows_v, t_c, agg_sh, isem, gsem, ssem):
    for j in range(3):
        pltpu.async_copy(eidx_s.at[j], ibuf.at[j], isem.at[j])
    pltpu.make_async_copy(eidx_s.at[0], ibuf.at[0], isem.at[0]).wait()
    pltpu.async_copy(t_c.at[ibuf.at[0, 0]], rows_v.at[0], gsem.at[0])

    @pl.loop(0, NGRPS - 1)
    def _(j):
        b = j % NBUF
        nb = (j + 1) % NBUF
        pltpu.make_async_copy(
            t_c.at[ibuf.at[b, 0]], rows_v.at[b], gsem.at[b]).wait()
        pltpu.async_copy(
            rows_v.at[b], agg_sh.at[ibuf.at[j % NIB, 1]], ssem.at[b],
            add=True)

        @pl.when(j > 0)
        def _():
            pltpu.make_async_copy(
                rows_v.at[nb], agg_sh.at[ibuf.at[0, 1]], ssem.at[nb]).wait()

        @pl.when(j + 3 < NGRPS)
        def _():
            pltpu.async_copy(eidx_s.at[j + 3], ibuf.at[(j + 3) % NIB],
                             isem.at[(j + 3) % NIB])

        pltpu.make_async_copy(
            eidx_s.at[0], ibuf.at[0], isem.at[(j + 1) % NIB]).wait()
        pltpu.async_copy(
            t_c.at[ibuf.at[(j + 1) % NIB, 0]], rows_v.at[nb], gsem.at[nb])

    jf = NGRPS - 1
    bf = jf % NBUF
    pltpu.make_async_copy(
        t_c.at[ibuf.at[bf, 0]], rows_v.at[bf], gsem.at[bf]).wait()
    pltpu.async_copy(
        rows_v.at[bf], agg_sh.at[ibuf.at[jf % NIB, 1]], ssem.at[bf],
        add=True)
    pltpu.make_async_copy(
        rows_v.at[1 - bf], agg_sh.at[ibuf.at[0, 1]], ssem.at[1 - bf]).wait()
    pltpu.make_async_copy(
        rows_v.at[bf], agg_sh.at[ibuf.at[0, 1]], ssem.at[bf]).wait()


def _make_body(with_deg):
    def body(*refs):
        if with_deg:
            (z_h, eidx, zeros64,
             out_h, bt_hbm,
             agg_sh, t_c, ibuf, rows_v, b_t, isem, gsem, ssem, dsem) = refs
        else:
            (z_h, eidx, zeros64, bt_hbm,
             out_h,
             agg_sh, t_c, ibuf, rows_v, b_t, isem, gsem, ssem, dsem) = refs
        c = lax.axis_index("c")
        s = lax.axis_index("s")
        eidx_s = eidx.at[s]
        aggb = rows_v.at[0, pl.ds(0, NROWCH), :]
        tb = rows_v.at[0, pl.ds(NROWCH, NROWCH), :]
        zb = rows_v.at[1, pl.ds(0, NROWCH), :]

        if with_deg:
            @pl.loop(0, NNCH)
            def _(ch):
                row0 = pl.multiple_of(s * NPT + ch * NROWCH, 8)
                pltpu.sync_copy(zeros64.at[pl.ds(row0, NROWCH), :],
                                agg_sh.at[pl.ds(row0, NROWCH), :])

            ones16v = jnp.ones((16,), jnp.float32)

            @pl.loop(0, KB * C)
            def _(r):
                for q in range(QN):
                    rows_v[0, r, pl.ds(q * 16, 16)] = ones16v

            plsc.subcore_barrier()

            @pl.loop(0, NGRPS // NIB)
            def _(u):
                pd = []
                for q in range(NIB):
                    pd.append(pltpu.async_copy(
                        eidx_s.at[u * NIB + q], ibuf.at[q], isem.at[q]))
                sd = []
                for q in range(NIB):
                    pd[q].wait()
                    sd.append(pltpu.async_copy(
                        rows_v.at[0], agg_sh.at[ibuf.at[q, 1]], dsem,
                        add=True))
                for d in sd:
                    d.wait()

            plsc.subcore_barrier()

            @pl.loop(0, NNCH)
            def _(ch):
                row0 = pl.multiple_of(s * NPT + ch * NROWCH, 8)
                pltpu.sync_copy(agg_sh.at[pl.ds(row0, NROWCH), :], aggb)

                @pl.loop(0, NROWCH)
                def _(r):
                    d = aggb[r, pl.ds(0, 16)] + 1.0
                    sq = jnp.where(d < 16.0, 1.0,
                                   jnp.where(d < 256.0, 4.0,
                                             jnp.where(d < 4096.0, 16.0,
                                                       64.0)))
                    sq = jnp.where(d < 65536.0, sq, 256.0)
                    for _ in range(8):
                        sq = 0.5 * (sq + d / sq)
                    b_t[ch * NROWCH + r, :] = 0.1 / sq

            pltpu.sync_copy(b_t, bt_hbm.at[c, s])
        else:
            pltpu.sync_copy(bt_hbm.at[c, s], b_t)

        @pl.loop(0, NNCH)
        def _(ch):
            row0 = pl.multiple_of(s * NPT + ch * NROWCH, 8)
            pltpu.sync_copy(z_h.at[c, s, pl.ds(ch * NROWCH, NROWCH), :], zb)

            @pl.loop(0, NROWCH)
            def _(r):
                cv = 10.0 * b_t[ch * NROWCH + r, :]
                for q in range(QN):
                    sl = pl.ds(q * 16, 16)
                    tb[r, sl] = cv * zb[r, sl]

            pltpu.sync_copy(tb, t_c.at[pl.ds(row0, NROWCH), :])
            pltpu.sync_copy(zeros64.at[pl.ds(row0, NROWCH), :],
                            agg_sh.at[pl.ds(row0, NROWCH), :])

        plsc.subcore_barrier()

        @pl.loop(0, K)
        def _(k):
            _edge_pipeline(eidx_s, ibuf, rows_v, t_c, agg_sh,
                           isem, gsem, ssem)
            plsc.subcore_barrier()

            zero16v = jnp.zeros((16,), jnp.float32)

            @pl.loop(0, NNCH)
            def _(ch):
                row0 = pl.multiple_of(s * NPT + ch * NROWCH, 8)
                d1 = pltpu.async_copy(
                    agg_sh.at[pl.ds(row0, NROWCH), :], aggb, gsem.at[0])
                d2 = pltpu.async_copy(
                    t_c.at[pl.ds(row0, NROWCH), :], tb, gsem.at[1])
                d3 = pltpu.async_copy(
                    z_h.at[c, s, pl.ds(ch * NROWCH, NROWCH), :], zb,
                    ssem.at[0])
                d1.wait()
                d2.wait()
                d3.wait()

                @pl.when(k < K - 1)
                def _():
                    @pl.loop(0, NROWCH, unroll=2)
                    def _(r):
                        bv = b_t[ch * NROWCH + r, :]
                        av = 90.0 * bv * bv
                        for q in range(QN):
                            sl = pl.ds(q * 16, 16)
                            tb[r, sl] = (av * (aggb[r, sl] + tb[r, sl])
                                         + bv * zb[r, sl])
                            aggb[r, sl] = zero16v

                    d4 = pltpu.async_copy(
                        tb, t_c.at[pl.ds(row0, NROWCH), :], gsem.at[0])
                    d5 = pltpu.async_copy(
                        aggb, agg_sh.at[pl.ds(row0, NROWCH), :], gsem.at[1])
                    d4.wait()
                    d5.wait()

                @pl.when(k == K - 1)
                def _():
                    @pl.loop(0, NROWCH, unroll=2)
                    def _(r):
                        bv = b_t[ch * NROWCH + r, :]
                        av = 90.0 * bv * bv
                        for q in range(QN):
                            sl = pl.ds(q * 16, 16)
                            tb[r, sl] = (av * (aggb[r, sl] + tb[r, sl])
                                         + bv * zb[r, sl]) / (10.0 * bv)

                    pltpu.async_copy(
                        tb, out_h.at[c, s, pl.ds(ch * NROWCH, NROWCH), :],
                        gsem.at[0]).wait()

            plsc.subcore_barrier()

    return body


_SCRATCH = [
    pltpu.VMEM_SHARED((NPAD, DH), jnp.float32),
    pltpu.VMEM_SHARED((NPAD, DH), jnp.float32),
    pltpu.VMEM((NIB, 2, KB * C), jnp.int32),
    pltpu.VMEM((NBUF, KB * C, DH), jnp.float32),
    pltpu.VMEM((NPT, 16), jnp.float32),
    pltpu.SemaphoreType.DMA((NIB,)),
    pltpu.SemaphoreType.DMA((NBUF,)),
    pltpu.SemaphoreType.DMA((NBUF,)),
    pltpu.SemaphoreType.DMA,
]

_sc_prop1 = functools.partial(
    pl.kernel,
    out_type=(
        jax.ShapeDtypeStruct((NCORE, NTEC, NPT, DH), jnp.float32),
        jax.ShapeDtypeStruct((NCORE, NTEC, NPT, 16), jnp.float32),
    ),
    mesh=plsc.VectorSubcoreMesh(core_axis_name="c", subcore_axis_name="s"),
    compiler_params=pltpu.CompilerParams(use_tc_tiling_on_sc=False),
    scratch_types=_SCRATCH,
)(_make_body(True))

_sc_prop2 = functools.partial(
    pl.kernel,
    out_type=jax.ShapeDtypeStruct((NCORE, NTEC, NPT, DH), jnp.float32),
    mesh=plsc.VectorSubcoreMesh(core_axis_name="c", subcore_axis_name="s"),
    compiler_params=pltpu.CompilerParams(use_tc_tiling_on_sc=False),
    scratch_types=_SCRATCH,
)(_make_body(False))


def _mlp_kernel(u_ref, w1_ref, b1_ref, w2_ref, b2_ref, o_ref):
    h = jnp.maximum(u_ref[...] @ w1_ref[...] + b1_ref[...], 0.0)
    o_ref[...] = h @ w2_ref[...] + b2_ref[...]


def _mlp(u, W1, b1, W2, b2, block_rows=1000):
    n, d_in = u.shape
    d_mid = W1.shape[1]
    d_out = W2.shape[1]
    return pl.pallas_call(
        _mlp_kernel,
        grid=(n // block_rows,),
        in_specs=[
            pl.BlockSpec((block_rows, d_in), lambda i: (i, 0)),
            pl.BlockSpec((d_in, d_mid), lambda i: (0, 0)),
            pl.BlockSpec((d_mid,), lambda i: (0,)),
            pl.BlockSpec((d_mid, d_out), lambda i: (0, 0)),
            pl.BlockSpec((d_out,), lambda i: (0,)),
        ],
        out_specs=pl.BlockSpec((block_rows, d_out), lambda i: (i, 0)),
        out_shape=jax.ShapeDtypeStruct((n, d_out), jnp.float32),
    )(u, W1, b1, W2, b2)


def _to_halves(z):
    zp = jnp.concatenate(
        [z, jnp.zeros((NPAD - N, z.shape[1]), jnp.float32)])
    return zp.reshape(NTEC, NPT, NCORE, DH).transpose(2, 0, 1, 3)


def _from_halves(z_h):
    return z_h.transpose(1, 2, 0, 3).reshape(NPAD, NCORE * DH)[:N]


def kernel(x, edge_index, W1, b1, W2, b2):
    src = edge_index[0].astype(jnp.int32)
    dst = edge_index[1].astype(jnp.int32)
    pad = jnp.full((NTEC * EPT - E,), N, jnp.int32)
    src_p = jnp.concatenate([src, pad]).reshape(NTEC, NGRPS, KB * C)
    dst_p = jnp.concatenate([dst, pad]).reshape(NTEC, NGRPS, KB * C)
    eidx = jnp.stack([src_p, dst_p], axis=2)

    zeros64 = jnp.zeros((NPAD, DH), jnp.float32)

    u_h, bt = _sc_prop1(_to_halves(x), eidx, zeros64)
    z2 = _mlp(_from_halves(u_h), W1, b1, W2, b2)
    out_h = _sc_prop2(_to_halves(z2), eidx, zeros64, bt)
    return _from_halves(out_h)


# speedup vs baseline: 1.0191x; 1.0003x over previous
import functools

import jax
import jax.numpy as jnp
from jax import lax
from jax.experimental import pallas as pl
from jax.experimental.pallas import tpu as pltpu
from jax.experimental.pallas import tpu_sc as plsc

N = 10000
E = 320000
K = 10

NCORE = 2
NTEC = 16
DH = 64
C = 128
KB = 2
NBUF = 2
NGRPS = 80
EPT = NGRPS * KB * C
NIB = 4
NPT = 640
NROWCH = 128
NNCH = NPT // NROWCH
NPAD = 10240
QN = DH // 16


def _edge_pipeline(eidx_s, ibuf, rows_v, t_c, agg_sh, isem, gsem, ssem):
    for j in range(3):
        pltpu.async_copy(eidx_s.at[j], ibuf.at[j], isem.at[j])
    pltpu.make_async_copy(eidx_s.at[0], ibuf.at[0], isem.at[0]).wait()
    pltpu.async_copy(t_c.at[ibuf.at[0, 0]], rows_v.at[0], gsem.at[0])

    @pl.loop(0, NGRPS - 1, unroll=2)
    def _(j):
        b = j % NBUF
        nb = (j + 1) % NBUF
        pltpu.make_async_copy(
            t_c.at[ibuf.at[b, 0]], rows_v.at[b], gsem.at[b]).wait()
        pltpu.async_copy(
            rows_v.at[b], agg_sh.at[ibuf.at[j % NIB, 1]], ssem.at[b],
            add=True)

        @pl.when(j > 0)
        def _():
            pltpu.make_async_copy(
                rows_v.at[nb], agg_sh.at[ibuf.at[0, 1]], ssem.at[nb]).wait()

        @pl.when(j + 3 < NGRPS)
        def _():
            pltpu.async_copy(eidx_s.at[j + 3], ibuf.at[(j + 3) % NIB],
                             isem.at[(j + 3) % NIB])

        pltpu.make_async_copy(
            eidx_s.at[0], ibuf.at[0], isem.at[(j + 1) % NIB]).wait()
        pltpu.async_copy(
            t_c.at[ibuf.at[(j + 1) % NIB, 0]], rows_v.at[nb], gsem.at[nb])

    jf = NGRPS - 1
    bf = jf % NBUF
    pltpu.make_async_copy(
        t_c.at[ibuf.at[bf, 0]], rows_v.at[bf], gsem.at[bf]).wait()
    pltpu.async_copy(
        rows_v.at[bf], agg_sh.at[ibuf.at[jf % NIB, 1]], ssem.at[bf],
        add=True)
    pltpu.make_async_copy(
        rows_v.at[1 - bf], agg_sh.at[ibuf.at[0, 1]], ssem.at[1 - bf]).wait()
    pltpu.make_async_copy(
        rows_v.at[bf], agg_sh.at[ibuf.at[0, 1]], ssem.at[bf]).wait()


def _make_body(with_deg):
    def body(*refs):
        if with_deg:
            (z_h, eidx, zeros64,
             out_h, bt_hbm,
             agg_sh, t_c, ibuf, rows_v, b_t, isem, gsem, ssem, dsem) = refs
        else:
            (z_h, eidx, zeros64, bt_hbm,
             out_h,
             agg_sh, t_c, ibuf, rows_v, b_t, isem, gsem, ssem, dsem) = refs
        c = lax.axis_index("c")
        s = lax.axis_index("s")
        eidx_s = eidx.at[s]
        aggb = rows_v.at[0, pl.ds(0, NROWCH), :]
        tb = rows_v.at[0, pl.ds(NROWCH, NROWCH), :]
        zb = rows_v.at[1, pl.ds(0, NROWCH), :]

        if with_deg:
            @pl.loop(0, NNCH)
            def _(ch):
                row0 = pl.multiple_of(s * NPT + ch * NROWCH, 8)
                pltpu.sync_copy(zeros64.at[pl.ds(row0, NROWCH), :],
                                agg_sh.at[pl.ds(row0, NROWCH), :])

            ones16v = jnp.ones((16,), jnp.float32)

            @pl.loop(0, KB * C)
            def _(r):
                for q in range(QN):
                    rows_v[0, r, pl.ds(q * 16, 16)] = ones16v

            plsc.subcore_barrier()

            @pl.loop(0, NGRPS // NIB)
            def _(u):
                pd = []
                for q in range(NIB):
                    pd.append(pltpu.async_copy(
                        eidx_s.at[u * NIB + q], ibuf.at[q], isem.at[q]))
                sd = []
                for q in range(NIB):
                    pd[q].wait()
                    sd.append(pltpu.async_copy(
                        rows_v.at[0], agg_sh.at[ibuf.at[q, 1]], dsem,
                        add=True))
                for d in sd:
                    d.wait()

            plsc.subcore_barrier()

            @pl.loop(0, NNCH)
            def _(ch):
                row0 = pl.multiple_of(s * NPT + ch * NROWCH, 8)
                pltpu.sync_copy(agg_sh.at[pl.ds(row0, NROWCH), :], aggb)

                @pl.loop(0, NROWCH)
                def _(r):
                    d = aggb[r, pl.ds(0, 16)] + 1.0
                    sq = jnp.where(d < 16.0, 1.0,
                                   jnp.where(d < 256.0, 4.0,
                                             jnp.where(d < 4096.0, 16.0,
                                                       64.0)))
                    sq = jnp.where(d < 65536.0, sq, 256.0)
                    for _ in range(8):
                        sq = 0.5 * (sq + d / sq)
                    b_t[ch * NROWCH + r, :] = 0.1 / sq

            pltpu.sync_copy(b_t, bt_hbm.at[c, s])
        else:
            pltpu.sync_copy(bt_hbm.at[c, s], b_t)

        @pl.loop(0, NNCH)
        def _(ch):
            row0 = pl.multiple_of(s * NPT + ch * NROWCH, 8)
            pltpu.sync_copy(z_h.at[c, s, pl.ds(ch * NROWCH, NROWCH), :], zb)

            @pl.loop(0, NROWCH)
            def _(r):
                cv = 10.0 * b_t[ch * NROWCH + r, :]
                for q in range(QN):
                    sl = pl.ds(q * 16, 16)
                    tb[r, sl] = cv * zb[r, sl]

            pltpu.sync_copy(tb, t_c.at[pl.ds(row0, NROWCH), :])
            pltpu.sync_copy(zeros64.at[pl.ds(row0, NROWCH), :],
                            agg_sh.at[pl.ds(row0, NROWCH), :])

        plsc.subcore_barrier()

        @pl.loop(0, K)
        def _(k):
            _edge_pipeline(eidx_s, ibuf, rows_v, t_c, agg_sh,
                           isem, gsem, ssem)
            plsc.subcore_barrier()

            zero16v = jnp.zeros((16,), jnp.float32)

            @pl.loop(0, NNCH)
            def _(ch):
                row0 = pl.multiple_of(s * NPT + ch * NROWCH, 8)
                d1 = pltpu.async_copy(
                    agg_sh.at[pl.ds(row0, NROWCH), :], aggb, gsem.at[0])
                d2 = pltpu.async_copy(
                    t_c.at[pl.ds(row0, NROWCH), :], tb, gsem.at[1])
                d3 = pltpu.async_copy(
                    z_h.at[c, s, pl.ds(ch * NROWCH, NROWCH), :], zb,
                    ssem.at[0])
                d1.wait()
                d2.wait()
                d3.wait()

                @pl.when(k < K - 1)
                def _():
                    @pl.loop(0, NROWCH, unroll=2)
                    def _(r):
                        bv = b_t[ch * NROWCH + r, :]
                        av = 90.0 * bv * bv
                        for q in range(QN):
                            sl = pl.ds(q * 16, 16)
                            tb[r, sl] = (av * (aggb[r, sl] + tb[r, sl])
                                         + bv * zb[r, sl])
                            aggb[r, sl] = zero16v

                    d4 = pltpu.async_copy(
                        tb, t_c.at[pl.ds(row0, NROWCH), :], gsem.at[0])
                    d5 = pltpu.async_copy(
                        aggb, agg_sh.at[pl.ds(row0, NROWCH), :], gsem.at[1])
                    d4.wait()
                    d5.wait()

                @pl.when(k == K - 1)
                def _():
                    @pl.loop(0, NROWCH, unroll=2)
                    def _(r):
                        bv = b_t[ch * NROWCH + r, :]
                        av = 90.0 * bv * bv
                        for q in range(QN):
                            sl = pl.ds(q * 16, 16)
                            tb[r, sl] = (av * (aggb[r, sl] + tb[r, sl])
                                         + bv * zb[r, sl]) / (10.0 * bv)

                    pltpu.async_copy(
                        tb, out_h.at[c, s, pl.ds(ch * NROWCH, NROWCH), :],
                        gsem.at[0]).wait()

            plsc.subcore_barrier()

    return body


_SCRATCH = [
    pltpu.VMEM_SHARED((NPAD, DH), jnp.float32),
    pltpu.VMEM_SHARED((NPAD, DH), jnp.float32),
    pltpu.VMEM((NIB, 2, KB * C), jnp.int32),
    pltpu.VMEM((NBUF, KB * C, DH), jnp.float32),
    pltpu.VMEM((NPT, 16), jnp.float32),
    pltpu.SemaphoreType.DMA((NIB,)),
    pltpu.SemaphoreType.DMA((NBUF,)),
    pltpu.SemaphoreType.DMA((NBUF,)),
    pltpu.SemaphoreType.DMA,
]

_sc_prop1 = functools.partial(
    pl.kernel,
    out_type=(
        jax.ShapeDtypeStruct((NCORE, NTEC, NPT, DH), jnp.float32),
        jax.ShapeDtypeStruct((NCORE, NTEC, NPT, 16), jnp.float32),
    ),
    mesh=plsc.VectorSubcoreMesh(core_axis_name="c", subcore_axis_name="s"),
    compiler_params=pltpu.CompilerParams(use_tc_tiling_on_sc=False),
    scratch_types=_SCRATCH,
)(_make_body(True))

_sc_prop2 = functools.partial(
    pl.kernel,
    out_type=jax.ShapeDtypeStruct((NCORE, NTEC, NPT, DH), jnp.float32),
    mesh=plsc.VectorSubcoreMesh(core_axis_name="c", subcore_axis_name="s"),
    compiler_params=pltpu.CompilerParams(use_tc_tiling_on_sc=False),
    scratch_types=_SCRATCH,
)(_make_body(False))


def _mlp_kernel(u_ref, w1_ref, b1_ref, w2_ref, b2_ref, o_ref):
    h = jnp.maximum(u_ref[...] @ w1_ref[...] + b1_ref[...], 0.0)
    o_ref[...] = h @ w2_ref[...] + b2_ref[...]


def _mlp(u, W1, b1, W2, b2, block_rows=1000):
    n, d_in = u.shape
    d_mid = W1.shape[1]
    d_out = W2.shape[1]
    return pl.pallas_call(
        _mlp_kernel,
        grid=(n // block_rows,),
        in_specs=[
            pl.BlockSpec((block_rows, d_in), lambda i: (i, 0)),
            pl.BlockSpec((d_in, d_mid), lambda i: (0, 0)),
            pl.BlockSpec((d_mid,), lambda i: (0,)),
            pl.BlockSpec((d_mid, d_out), lambda i: (0, 0)),
            pl.BlockSpec((d_out,), lambda i: (0,)),
        ],
        out_specs=pl.BlockSpec((block_rows, d_out), lambda i: (i, 0)),
        out_shape=jax.ShapeDtypeStruct((n, d_out), jnp.float32),
    )(u, W1, b1, W2, b2)


def _to_halves(z):
    zp = jnp.concatenate(
        [z, jnp.zeros((NPAD - N, z.shape[1]), jnp.float32)])
    return zp.reshape(NTEC, NPT, NCORE, DH).transpose(2, 0, 1, 3)


def _from_halves(z_h):
    return z_h.transpose(1, 2, 0, 3).reshape(NPAD, NCORE * DH)[:N]


def kernel(x, edge_index, W1, b1, W2, b2):
    src = edge_index[0].astype(jnp.int32)
    dst = edge_index[1].astype(jnp.int32)
    pad = jnp.full((NTEC * EPT - E,), N, jnp.int32)
    src_p = jnp.concatenate([src, pad]).reshape(NTEC, NGRPS, KB * C)
    dst_p = jnp.concatenate([dst, pad]).reshape(NTEC, NGRPS, KB * C)
    eidx = jnp.stack([src_p, dst_p], axis=2)

    zeros64 = jnp.zeros((NPAD, DH), jnp.float32)

    u_h, bt = _sc_prop1(_to_halves(x), eidx, zeros64)
    z2 = _mlp(_from_halves(u_h), W1, b1, W2, b2)
    out_h = _sc_prop2(_to_halves(z2), eidx, zeros64, bt)
    return _from_halves(out_h)


# speedup vs baseline: 1.0373x; 1.0179x over previous
import functools

import jax
import jax.numpy as jnp
from jax import lax
from jax.experimental import pallas as pl
from jax.experimental.pallas import tpu as pltpu
from jax.experimental.pallas import tpu_sc as plsc

N = 10000
E = 320000
K = 10

NCORE = 2
NTEC = 16
DH = 64
C = 128
KB = 2
NBUF = 2
NGRPS = 80
EPT = NGRPS * KB * C
NIB = 4
NPT = 640
NROWCH = 128
NNCH = NPT // NROWCH
NPAD = 10240
QN = DH // 16


def _edge_pipeline(eidx_s, ibuf, rows_v, t_c, agg_sh, isem, gsem, ssem):
    for j in range(3):
        pltpu.async_copy(eidx_s.at[j], ibuf.at[j], isem.at[j])
    pltpu.make_async_copy(eidx_s.at[0], ibuf.at[0], isem.at[0]).wait()
    pltpu.async_copy(t_c.at[ibuf.at[0, 0]], rows_v.at[0], gsem.at[0])

    @pl.loop(0, NGRPS - 1, unroll=2)
    def _(j):
        b = j % NBUF
        nb = (j + 1) % NBUF
        pltpu.make_async_copy(
            t_c.at[ibuf.at[b, 0]], rows_v.at[b], gsem.at[b]).wait()
        pltpu.async_copy(
            rows_v.at[b], agg_sh.at[ibuf.at[j % NIB, 1]], ssem.at[b],
            add=True)

        @pl.when(j > 0)
        def _():
            pltpu.make_async_copy(
                rows_v.at[nb], agg_sh.at[ibuf.at[0, 1]], ssem.at[nb]).wait()

        @pl.when(j + 3 < NGRPS)
        def _():
            pltpu.async_copy(eidx_s.at[j + 3], ibuf.at[(j + 3) % NIB],
                             isem.at[(j + 3) % NIB])

        pltpu.make_async_copy(
            eidx_s.at[0], ibuf.at[0], isem.at[(j + 1) % NIB]).wait()
        pltpu.async_copy(
            t_c.at[ibuf.at[(j + 1) % NIB, 0]], rows_v.at[nb], gsem.at[nb])

    jf = NGRPS - 1
    bf = jf % NBUF
    pltpu.make_async_copy(
        t_c.at[ibuf.at[bf, 0]], rows_v.at[bf], gsem.at[bf]).wait()
    pltpu.async_copy(
        rows_v.at[bf], agg_sh.at[ibuf.at[jf % NIB, 1]], ssem.at[bf],
        add=True)
    pltpu.make_async_copy(
        rows_v.at[1 - bf], agg_sh.at[ibuf.at[0, 1]], ssem.at[1 - bf]).wait()
    pltpu.make_async_copy(
        rows_v.at[bf], agg_sh.at[ibuf.at[0, 1]], ssem.at[bf]).wait()


def _make_body(with_deg):
    def body(*refs):
        if with_deg:
            (z_h, eidx, zeros64,
             out_h, bt_hbm,
             agg_sh, t_c, ibuf, rows_v, b_t, isem, gsem, ssem, dsem) = refs
        else:
            (z_h, eidx, zeros64, bt_hbm,
             out_h,
             agg_sh, t_c, ibuf, rows_v, b_t, isem, gsem, ssem, dsem) = refs
        c = lax.axis_index("c")
        s = lax.axis_index("s")
        eidx_s = eidx.at[s]
        aggb = rows_v.at[0, pl.ds(0, NROWCH), :]
        tb = rows_v.at[0, pl.ds(NROWCH, NROWCH), :]
        zb = rows_v.at[1, pl.ds(0, NROWCH), :]

        if with_deg:
            @pl.loop(0, NNCH)
            def _(ch):
                row0 = pl.multiple_of(s * NPT + ch * NROWCH, 8)
                pltpu.sync_copy(zeros64.at[pl.ds(row0, NROWCH), :],
                                agg_sh.at[pl.ds(row0, NROWCH), :])

            ones16v = jnp.ones((16,), jnp.float32)

            @pl.loop(0, KB * C)
            def _(r):
                for q in range(QN):
                    rows_v[0, r, pl.ds(q * 16, 16)] = ones16v

            plsc.subcore_barrier()

            @pl.loop(0, NGRPS // NIB)
            def _(u):
                pd = []
                for q in range(NIB):
                    pd.append(pltpu.async_copy(
                        eidx_s.at[u * NIB + q], ibuf.at[q], isem.at[q]))
                sd = []
                for q in range(NIB):
                    pd[q].wait()
                    sd.append(pltpu.async_copy(
                        rows_v.at[0], agg_sh.at[ibuf.at[q, 1]], dsem,
                        add=True))
                for d in sd:
                    d.wait()

            plsc.subcore_barrier()

            @pl.loop(0, NNCH)
            def _(ch):
                row0 = pl.multiple_of(s * NPT + ch * NROWCH, 8)
                pltpu.sync_copy(agg_sh.at[pl.ds(row0, NROWCH), :], aggb)

                @pl.loop(0, NROWCH)
                def _(r):
                    d = aggb[r, pl.ds(0, 16)] + 1.0
                    sq = jnp.where(d < 16.0, 1.0,
                                   jnp.where(d < 256.0, 4.0,
                                             jnp.where(d < 4096.0, 16.0,
                                                       64.0)))
                    sq = jnp.where(d < 65536.0, sq, 256.0)
                    for _ in range(8):
                        sq = 0.5 * (sq + d / sq)
                    b_t[ch * NROWCH + r, :] = 0.1 / sq

            pltpu.sync_copy(b_t, bt_hbm.at[c, s])
        else:
            pltpu.sync_copy(bt_hbm.at[c, s], b_t)

        @pl.loop(0, NNCH)
        def _(ch):
            row0 = pl.multiple_of(s * NPT + ch * NROWCH, 8)
            pltpu.sync_copy(z_h.at[c, s, pl.ds(ch * NROWCH, NROWCH), :], zb)

            @pl.loop(0, NROWCH)
            def _(r):
                cv = 10.0 * b_t[ch * NROWCH + r, :]
                for q in range(QN):
                    sl = pl.ds(q * 16, 16)
                    tb[r, sl] = cv * zb[r, sl]

            pltpu.sync_copy(tb, t_c.at[pl.ds(row0, NROWCH), :])
            pltpu.sync_copy(zeros64.at[pl.ds(row0, NROWCH), :],
                            agg_sh.at[pl.ds(row0, NROWCH), :])

        plsc.subcore_barrier()

        @pl.loop(0, K)
        def _(k):
            _edge_pipeline(eidx_s, ibuf, rows_v, t_c, agg_sh,
                           isem, gsem, ssem)
            plsc.subcore_barrier()

            zero16v = jnp.zeros((16,), jnp.float32)

            @pl.loop(0, NNCH)
            def _(ch):
                row0 = pl.multiple_of(s * NPT + ch * NROWCH, 8)
                d1 = pltpu.async_copy(
                    agg_sh.at[pl.ds(row0, NROWCH), :], aggb, gsem.at[0])
                d2 = pltpu.async_copy(
                    t_c.at[pl.ds(row0, NROWCH), :], tb, gsem.at[1])
                d3 = pltpu.async_copy(
                    z_h.at[c, s, pl.ds(ch * NROWCH, NROWCH), :], zb,
                    ssem.at[0])
                d1.wait()
                d2.wait()
                d3.wait()

                @pl.when(k < K - 1)
                def _():
                    @pl.loop(0, NROWCH, unroll=2)
                    def _(r):
                        bv = b_t[ch * NROWCH + r, :]
                        av = 90.0 * bv * bv
                        for q in range(QN):
                            sl = pl.ds(q * 16, 16)
                            tb[r, sl] = (av * (aggb[r, sl] + tb[r, sl])
                                         + bv * zb[r, sl])
                            aggb[r, sl] = zero16v

                    d4 = pltpu.async_copy(
                        tb, t_c.at[pl.ds(row0, NROWCH), :], gsem.at[0])
                    d5 = pltpu.async_copy(
                        aggb, agg_sh.at[pl.ds(row0, NROWCH), :], gsem.at[1])
                    d4.wait()
                    d5.wait()

                @pl.when(k == K - 1)
                def _():
                    @pl.loop(0, NROWCH, unroll=2)
                    def _(r):
                        bv = b_t[ch * NROWCH + r, :]
                        av = 90.0 * bv * bv
                        for q in range(QN):
                            sl = pl.ds(q * 16, 16)
                            tb[r, sl] = (av * (aggb[r, sl] + tb[r, sl])
                                         + bv * zb[r, sl]) / (10.0 * bv)

                    pltpu.async_copy(
                        tb, out_h.at[c, s, pl.ds(ch * NROWCH, NROWCH), :],
                        gsem.at[0]).wait()

            plsc.subcore_barrier()

    return body


_SCRATCH = [
    pltpu.VMEM_SHARED((NPAD, DH), jnp.float32),
    pltpu.VMEM_SHARED((NPAD, DH), jnp.float32),
    pltpu.VMEM((NIB, 2, KB * C), jnp.int32),
    pltpu.VMEM((NBUF, KB * C, DH), jnp.float32),
    pltpu.VMEM((NPT, 16), jnp.float32),
    pltpu.SemaphoreType.DMA((NIB,)),
    pltpu.SemaphoreType.DMA((NBUF,)),
    pltpu.SemaphoreType.DMA((NBUF,)),
    pltpu.SemaphoreType.DMA,
]

_sc_prop1 = functools.partial(
    pl.kernel,
    out_type=(
        jax.ShapeDtypeStruct((NCORE, NTEC, NPT, DH), jnp.float32),
        jax.ShapeDtypeStruct((NCORE, NTEC, NPT, 16), jnp.float32),
    ),
    mesh=plsc.VectorSubcoreMesh(core_axis_name="c", subcore_axis_name="s"),
    compiler_params=pltpu.CompilerParams(use_tc_tiling_on_sc=False),
    scratch_types=_SCRATCH,
)(_make_body(True))

_sc_prop2 = functools.partial(
    pl.kernel,
    out_type=jax.ShapeDtypeStruct((NCORE, NTEC, NPT, DH), jnp.float32),
    mesh=plsc.VectorSubcoreMesh(core_axis_name="c", subcore_axis_name="s"),
    compiler_params=pltpu.CompilerParams(use_tc_tiling_on_sc=False),
    scratch_types=_SCRATCH,
)(_make_body(False))


def _mlp_kernel(ua_ref, ub_ref, w1_ref, b1_ref, w2_ref, b2_ref, o_ref):
    u = jnp.concatenate([ua_ref[0, 0], ub_ref[0, 0]], axis=-1)
    h = jnp.maximum(u @ w1_ref[...] + b1_ref[...], 0.0)
    z2 = h @ w2_ref[...] + b2_ref[...]
    o_ref[0, 0] = z2[:, :DH]
    o_ref[1, 0] = z2[:, DH:]


def _mlp(u_h, W1, b1, W2, b2):
    d_mid = W1.shape[1]
    d_out = W2.shape[1]
    return pl.pallas_call(
        _mlp_kernel,
        grid=(NTEC,),
        in_specs=[
            pl.BlockSpec((1, 1, NPT, DH), lambda i: (0, i, 0, 0)),
            pl.BlockSpec((1, 1, NPT, DH), lambda i: (1, i, 0, 0)),
            pl.BlockSpec((NCORE * DH, d_mid), lambda i: (0, 0)),
            pl.BlockSpec((d_mid,), lambda i: (0,)),
            pl.BlockSpec((d_mid, d_out), lambda i: (0, 0)),
            pl.BlockSpec((d_out,), lambda i: (0,)),
        ],
        out_specs=pl.BlockSpec((NCORE, 1, NPT, DH), lambda i: (0, i, 0, 0)),
        out_shape=jax.ShapeDtypeStruct((NCORE, NTEC, NPT, DH), jnp.float32),
    )(u_h, u_h, W1, b1, W2, b2)


def _to_halves(z):
    zp = jnp.concatenate(
        [z, jnp.zeros((NPAD - N, z.shape[1]), jnp.float32)])
    return zp.reshape(NTEC, NPT, NCORE, DH).transpose(2, 0, 1, 3)


def _from_halves(z_h):
    return z_h.transpose(1, 2, 0, 3).reshape(NPAD, NCORE * DH)[:N]


def kernel(x, edge_index, W1, b1, W2, b2):
    src = edge_index[0].astype(jnp.int32)
    dst = edge_index[1].astype(jnp.int32)
    pad = jnp.full((NTEC * EPT - E,), N, jnp.int32)
    src_p = jnp.concatenate([src, pad]).reshape(NTEC, NGRPS, KB * C)
    dst_p = jnp.concatenate([dst, pad]).reshape(NTEC, NGRPS, KB * C)
    eidx = jnp.stack([src_p, dst_p], axis=2)

    zeros64 = jnp.zeros((NPAD, DH), jnp.float32)

    u_h, bt = _sc_prop1(_to_halves(x), eidx, zeros64)
    z2_h = _mlp(u_h, W1, b1, W2, b2)
    out_h = _sc_prop2(z2_h, eidx, zeros64, bt)
    return _from_halves(out_h)
